# baseline (speedup 1.0000x reference)
import jax
import jax.numpy as jnp
from jax.experimental import pallas as pl
from jax.experimental.pallas import tpu as pltpu

_LABELNUM = 1000
_BBLK = 512


def _onehot_block(idx_ref, out_ref):
    idx = idx_ref[...]
    rows = jax.lax.broadcasted_iota(jnp.int32, out_ref.shape, 0)
    out_ref[...] = (rows == idx[0]).astype(jnp.float32)


def kernel(input):
    B = input.shape[0]
    nblk = B // _BBLK
    idx3 = input.astype(jnp.int32).reshape(nblk, 1, _BBLK)
    out_t = pl.pallas_call(
        _onehot_block,
        grid=(nblk,),
        in_specs=[pl.BlockSpec((1, 1, _BBLK), lambda i: (i, 0, 0))],
        out_specs=pl.BlockSpec((_LABELNUM, _BBLK), lambda i: (0, i)),
        out_shape=jax.ShapeDtypeStruct((_LABELNUM, B), jnp.float32),
        compiler_params=pltpu.CompilerParams(
            dimension_semantics=("parallel",),
        ),
    )(idx3)
    return out_t.T


# speedup vs baseline: 1.3058x; 1.3058x over previous
import jax
import jax.numpy as jnp
from jax.experimental import pallas as pl
from jax.experimental.pallas import tpu as pltpu

_LABELNUM = 1000
_BBLK = 1024


def _onehot_block(idx_ref, out_ref):
    idx = idx_ref[...]
    rows = jax.lax.broadcasted_iota(jnp.int32, out_ref.shape, 0)
    out_ref[...] = jnp.where(rows == idx[0], 1.0, 0.0).astype(jnp.float32)


def kernel(input):
    B = input.shape[0]
    nblk = B // _BBLK
    idx3 = input.astype(jnp.int32).reshape(nblk, 1, _BBLK)
    out_t = pl.pallas_call(
        _onehot_block,
        grid=(nblk,),
        in_specs=[pl.BlockSpec((1, 1, _BBLK), lambda i: (i, 0, 0))],
        out_specs=pl.BlockSpec((_LABELNUM, _BBLK), lambda i: (0, i)),
        out_shape=jax.ShapeDtypeStruct((_LABELNUM, B), jnp.float32),
        compiler_params=pltpu.CompilerParams(
            dimension_semantics=("parallel",),
        ),
    )(idx3)
    return out_t.T
